# baseline (speedup 1.0000x reference)
import functools

import jax
import jax.numpy as jnp
from jax import lax
from jax.experimental import pallas as pl
from jax.experimental.pallas import tpu as pltpu
from jax.experimental.pallas import tpu_sc as plsc

VOCAB = 1_000_000
HALF = VOCAB // 2
D = 64
BATCH = 16384
HIST = 50
B = BATCH * HIST

NC = 2
NS = 16
NW = NC * NS

U = 128
NB = BATCH // U
NUNIT = HIST * NB
UPW = NUNIT // NW


_W = 4096
_TGRID = -(-VOCAB // _W)
_PROWS = _TGRID * (_W // 2)


def _relayout_body(tin_ref, tout_ref):
    a = tin_ref[...]
    tout_ref[...] = jnp.concatenate(
        [a[:, : _W // 2].T, a[:, _W // 2 :].T], axis=1
    )


_relayout = pl.pallas_call(
    _relayout_body,
    grid=(_TGRID,),
    in_specs=[pl.BlockSpec((D, _W), lambda i: (0, i))],
    out_specs=pl.BlockSpec((_W // 2, 2 * D), lambda i: (i, 0)),
    out_shape=jax.ShapeDtypeStruct((_PROWS, 2 * D), jnp.float32),
)


_mesh = plsc.VectorSubcoreMesh(core_axis_name="c", subcore_axis_name="s")


@functools.partial(
    pl.kernel,
    mesh=_mesh,
    out_type=jax.ShapeDtypeStruct((HIST, 8, NB, 8 * U), jnp.float32),
    scratch_types=[
        pltpu.VMEM((2 * U,), jnp.int32),
        pltpu.VMEM((2 * U, D), jnp.float32),
        pltpu.VMEM((D * U,), jnp.float32),
        pltpu.SemaphoreType.DMA,
        pltpu.SemaphoreType.DMA,
    ],
    compiler_params=pltpu.CompilerParams(
        use_tc_tiling_on_sc=False, needs_layout_passes=False
    ),
)
def _emb_gather(tlin_hbm, xg_hbm, out_hbm, idx_v, g_v, tr_v, gsem, osem):
    wid = lax.axis_index("s") * NC + lax.axis_index("c")

    iota16 = lax.iota(jnp.int32, 16)
    row_base = [iota16 + (j * 16) for j in range(8)]

    def fire(t, buf):
        u = wid + NW * t
        h = u // NB
        b0 = (u % NB) * U
        pltpu.sync_copy(xg_hbm.at[h, pl.ds(b0, U)], idx_v.at[pl.ds(buf * U, U)])
        pltpu.async_copy(
            tlin_hbm.at[idx_v.at[pl.ds(buf * U, U)]],
            g_v.at[pl.ds(buf * U, U)],
            gsem,
        )

    def wait_gather(buf):
        pltpu.make_async_copy(
            tlin_hbm.at[pl.ds(0, U)],
            g_v.at[pl.ds(buf * U, U)],
            gsem,
        ).wait()

    def wait_out():
        for _ in range(8):
            pltpu.make_async_copy(
                tr_v.at[pl.ds(0, 8 * U)],
                out_hbm.at[0, 0, 0, pl.ds(0, 8 * U)],
                osem,
            ).wait()

    fire(0, 0)

    def body(t, carry):
        buf = lax.rem(t, 2)
        pbuf = 1 - buf

        @pl.when(t < UPW)
        def _():
            fire(t, buf)

        @pl.when(t >= 2)
        def _():
            wait_out()

        wait_gather(pbuf)

        robase = pbuf * U
        for j in range(8):
            rows_j = row_base[j] + robase
            for d in range(64):
                col = jnp.full((16,), d, jnp.int32)
                vals = plsc.load_gather(g_v, [rows_j, col])
                tr_v[pl.ds(d * U + j * 16, 16)] = vals

        u = wid + NW * (t - 1)
        h = u // NB
        bb = u % NB
        for dh in range(8):
            pltpu.async_copy(
                tr_v.at[pl.ds(dh * 8 * U, 8 * U)],
                out_hbm.at[h, dh, bb, pl.ds(0, 8 * U)],
                osem,
            )
        return carry

    lax.fori_loop(1, UPW + 1, body, 0)
    wait_out()


def kernel(x, table):
    tpack = _relayout(jnp.swapaxes(table, 0, 1))
    tlin = tpack.reshape(2 * _PROWS, D)
    xT = jnp.swapaxes(x, 0, 1)
    xg = ((xT >> 12) << 12) + 2 * (xT & 2047) + ((xT >> 11) & 1)
    out5 = _emb_gather(tlin, xg)
    out = out5.reshape(HIST, 8, NB, 8, U).transpose(2, 4, 0, 1, 3)
    return out.reshape(BATCH, HIST, D)


# speedup vs baseline: 1.4206x; 1.4206x over previous
import functools

import jax
import jax.numpy as jnp
from jax import lax
from jax.experimental import pallas as pl
from jax.experimental.pallas import tpu as pltpu
from jax.experimental.pallas import tpu_sc as plsc

VOCAB = 1_000_000
D = 64
BATCH = 16384
HIST = 50
B = BATCH * HIST

NC = 2
NS = 16
NW = NC * NS
BPW = B // NW

IPS = 128
K = 5
CH = K * IPS
NCHUNK = BPW // CH
NBUF = 2


_W = 4096
_TGRID = -(-VOCAB // _W)
_PROWS = _TGRID * (_W // 2)


def _relayout_body(tin_ref, tout_ref):
    a = tin_ref[...]
    tout_ref[...] = jnp.concatenate(
        [a[:, : _W // 2].T, a[:, _W // 2 :].T], axis=1
    )


_relayout = pl.pallas_call(
    _relayout_body,
    grid=(_TGRID,),
    in_specs=[pl.BlockSpec((D, _W), lambda i: (0, i))],
    out_specs=pl.BlockSpec((_W // 2, 2 * D), lambda i: (i, 0)),
    out_shape=jax.ShapeDtypeStruct((_PROWS, 2 * D), jnp.float32),
)


_mesh = plsc.VectorSubcoreMesh(core_axis_name="c", subcore_axis_name="s")


@functools.partial(
    pl.kernel,
    mesh=_mesh,
    out_type=jax.ShapeDtypeStruct((B, D), jnp.float32),
    scratch_types=[
        pltpu.VMEM((NBUF * CH,), jnp.int32),
        pltpu.VMEM((NBUF * CH, D), jnp.float32),
        pltpu.SemaphoreType.DMA,
        pltpu.SemaphoreType.DMA,
    ],
    compiler_params=pltpu.CompilerParams(use_tc_tiling_on_sc=False),
)
def _emb_gather(idx_hbm, tlin_hbm, out_hbm, idx_v, rows_v, gsem, osem):
    wid = lax.axis_index("s") * NC + lax.axis_index("c")
    base = wid * BPW

    def fire(g, b):
        off = base + g * CH
        pltpu.sync_copy(idx_hbm.at[pl.ds(off, CH)], idx_v.at[pl.ds(b * CH, CH)])
        for j in range(K):
            pltpu.async_copy(
                tlin_hbm.at[idx_v.at[pl.ds(b * CH + j * IPS, IPS)]],
                rows_v.at[pl.ds(b * CH + j * IPS, IPS)],
                gsem,
            )

    def wait_gathers(b):
        pltpu.make_async_copy(
            out_hbm.at[pl.ds(base, CH)], rows_v.at[pl.ds(b * CH, CH)], gsem
        ).wait()

    def store(g, b):
        off = base + g * CH
        pltpu.async_copy(
            rows_v.at[pl.ds(b * CH, CH)], out_hbm.at[pl.ds(off, CH)], osem
        )

    def wait_store():
        pltpu.make_async_copy(
            rows_v.at[pl.ds(0, CH)], out_hbm.at[pl.ds(base, CH)], osem
        ).wait()

    fire(0, 0)

    def body(g, carry):
        b = lax.rem(g, NBUF)
        pb = 1 - b

        @pl.when(g >= 2)
        def _():
            wait_store()

        fire(g, b)
        wait_gathers(pb)
        store(g - 1, pb)
        return carry

    lax.fori_loop(1, NCHUNK, body, 0)

    last = NCHUNK - 1
    wait_gathers(last % NBUF)
    store(last, last % NBUF)
    wait_store()
    wait_store()


_BB = 2048
_NBB = BATCH // _BB


def _fold_body(lin_ref, out_ref):
    out_ref[...] = lin_ref[...].T[None, :, :]


_fold = pl.pallas_call(
    _fold_body,
    grid=(HIST, _NBB),
    in_specs=[pl.BlockSpec((_BB, D), lambda h, b: (h * _NBB + b, 0))],
    out_specs=pl.BlockSpec((1, D, _BB), lambda h, b: (h, 0, b)),
    out_shape=jax.ShapeDtypeStruct((HIST, D, BATCH), jnp.float32),
)


def kernel(x, table):
    tpack = _relayout(jnp.swapaxes(table, 0, 1))
    tlin = tpack.reshape(2 * _PROWS, D)
    xT = jnp.swapaxes(x, 0, 1)
    xg = ((xT >> 12) << 12) + 2 * (xT & 2047) + ((xT >> 11) & 1)
    lin = _emb_gather(xg.reshape(B), tlin)
    out_phys = _fold(lin)
    return jnp.transpose(out_phys, (2, 0, 1))


# speedup vs baseline: 1.5370x; 1.0819x over previous
import functools

import jax
import jax.numpy as jnp
from jax import lax
from jax.experimental import pallas as pl
from jax.experimental.pallas import tpu as pltpu
from jax.experimental.pallas import tpu_sc as plsc

VOCAB = 1_000_000
D = 64
BATCH = 16384
HIST = 50
B = BATCH * HIST

NC = 2
NS = 16
NW = NC * NS
BPW = B // NW

IPS = 128
K = 5
CH = K * IPS
NCHUNK = BPW // CH
NBUF = 2


_W = 4096
_TGRID = -(-VOCAB // _W)
_PROWS = _TGRID * (_W // 2)


def _relayout_body(tin_ref, tout_ref):
    a = tin_ref[...]
    tout_ref[...] = jnp.concatenate(
        [a[:, : _W // 2].T, a[:, _W // 2 :].T], axis=1
    )


_relayout = pl.pallas_call(
    _relayout_body,
    grid=(_TGRID,),
    in_specs=[pl.BlockSpec((D, _W), lambda i: (0, i))],
    out_specs=pl.BlockSpec((_W // 2, 2 * D), lambda i: (i, 0)),
    out_shape=jax.ShapeDtypeStruct((_PROWS, 2 * D), jnp.float32),
)


_mesh = plsc.VectorSubcoreMesh(core_axis_name="c", subcore_axis_name="s")


@functools.partial(
    pl.kernel,
    mesh=_mesh,
    out_type=jax.ShapeDtypeStruct((B, D), jnp.float32),
    scratch_types=[
        pltpu.VMEM((NBUF * CH,), jnp.int32),
        pltpu.VMEM((NBUF * CH, D), jnp.float32),
        pltpu.SemaphoreType.DMA,
        pltpu.SemaphoreType.DMA,
    ],
    compiler_params=pltpu.CompilerParams(use_tc_tiling_on_sc=False),
)
def _emb_gather(idx_hbm, tlin_hbm, out_hbm, idx_v, rows_v, gsem, osem):
    wid = lax.axis_index("s") * NC + lax.axis_index("c")
    base = wid * BPW

    def fire(g, b):
        off = base + g * CH
        pltpu.sync_copy(idx_hbm.at[pl.ds(off, CH)], idx_v.at[pl.ds(b * CH, CH)])
        for j in range(K):
            pltpu.async_copy(
                tlin_hbm.at[idx_v.at[pl.ds(b * CH + j * IPS, IPS)]],
                rows_v.at[pl.ds(b * CH + j * IPS, IPS)],
                gsem,
            )

    def wait_gathers(b):
        pltpu.make_async_copy(
            out_hbm.at[pl.ds(base, CH)], rows_v.at[pl.ds(b * CH, CH)], gsem
        ).wait()

    def store(g, b):
        off = base + g * CH
        pltpu.async_copy(
            rows_v.at[pl.ds(b * CH, CH)], out_hbm.at[pl.ds(off, CH)], osem
        )

    def wait_store():
        pltpu.make_async_copy(
            rows_v.at[pl.ds(0, CH)], out_hbm.at[pl.ds(base, CH)], osem
        ).wait()

    fire(0, 0)

    def body(g, carry):
        b = lax.rem(g, NBUF)
        pb = 1 - b

        @pl.when(g >= 2)
        def _():
            wait_store()

        fire(g, b)
        wait_gathers(pb)
        store(g - 1, pb)
        return carry

    lax.fori_loop(1, NCHUNK, body, 0)

    last = NCHUNK - 1
    wait_gathers(last % NBUF)
    store(last, last % NBUF)
    wait_store()
    wait_store()


_BB = 2048
_NBB = BATCH // _BB


def _fold_body(lin_ref, out_ref):
    a = lin_ref[...]
    c = jnp.concatenate([a[:, :D], a[:, D:]], axis=0)
    out_ref[...] = c.T[None, :, :]


_fold = pl.pallas_call(
    _fold_body,
    grid=(HIST, _NBB),
    in_specs=[pl.BlockSpec((_BB // 2, 2 * D), lambda h, b: (h * _NBB + b, 0))],
    out_specs=pl.BlockSpec((1, D, _BB), lambda h, b: (h, 0, b)),
    out_shape=jax.ShapeDtypeStruct((HIST, D, BATCH), jnp.float32),
)


def kernel(x, table):
    tpack = _relayout(jnp.swapaxes(table, 0, 1))
    tlin = tpack.reshape(2 * _PROWS, D)
    xT = jnp.swapaxes(x, 0, 1)
    xg = ((xT >> 12) << 12) + 2 * (xT & 2047) + ((xT >> 11) & 1)
    xgp = (
        xg.reshape(HIST, _NBB, 2, _BB // 2)
        .transpose(0, 1, 3, 2)
        .reshape(B)
    )
    lin = _emb_gather(xgp, tlin)
    out_phys = _fold(lin.reshape(B // 2, 2 * D))
    return jnp.transpose(out_phys, (2, 0, 1))


# speedup vs baseline: 1.9097x; 1.2425x over previous
import functools

import jax
import jax.numpy as jnp
from jax import lax
from jax.experimental import pallas as pl
from jax.experimental.pallas import tpu as pltpu
from jax.experimental.pallas import tpu_sc as plsc

VOCAB = 1_000_000
D = 64
BATCH = 16384
HIST = 50
B = BATCH * HIST

NC = 2
NS = 16
NW = NC * NS
BPW = B // NW

IPS = 128
K = 4
CH = K * IPS
NCHUNK = BPW // CH
NBUF = 2
FB = 2048


_W = 4096
_TGRID = -(-VOCAB // _W)
_PROWS = _TGRID * (_W // 2)


def _relayout_body(tin_ref, tout_ref):
    a = tin_ref[...]
    tout_ref[...] = jnp.concatenate(
        [a[:, : _W // 2].T, a[:, _W // 2 :].T], axis=1
    )


_relayout = pl.pallas_call(
    _relayout_body,
    grid=(_TGRID,),
    in_specs=[pl.BlockSpec((D, _W), lambda i: (0, i))],
    out_specs=pl.BlockSpec((_W // 2, 2 * D), lambda i: (i, 0)),
    out_shape=jax.ShapeDtypeStruct((_PROWS, 2 * D), jnp.float32),
)


_mesh = plsc.VectorSubcoreMesh(core_axis_name="c", subcore_axis_name="s")


@functools.partial(
    pl.kernel,
    mesh=_mesh,
    out_type=jax.ShapeDtypeStruct((B, D), jnp.float32),
    scratch_types=[
        pltpu.VMEM((NBUF * CH,), jnp.int32),
        pltpu.VMEM((CH,), jnp.int32),
        pltpu.VMEM((NBUF * CH, D), jnp.float32),
        pltpu.SemaphoreType.DMA,
        pltpu.SemaphoreType.DMA,
    ],
    compiler_params=pltpu.CompilerParams(
        use_tc_tiling_on_sc=False, needs_layout_passes=False
    ),
)
def _emb_gather(idx_hbm, tlin_hbm, out_hbm, idx_v, idx_s, rows_v, gsem, osem):
    wid = lax.axis_index("s") * NC + lax.axis_index("c")
    base = wid * BPW
    iota2 = lax.iota(jnp.int32, 16) * 2

    def fire(g, b):
        off = base + g * CH
        blk = off // FB
        pos = lax.rem(off, FB) // 2
        srcA = pl.multiple_of(blk * FB + pos, 256)
        pltpu.sync_copy(idx_hbm.at[pl.ds(srcA, CH // 2)], idx_s.at[pl.ds(0, CH // 2)])
        pltpu.sync_copy(
            idx_hbm.at[pl.ds(srcA + FB // 2, CH // 2)],
            idx_s.at[pl.ds(CH // 2, CH // 2)],
        )
        for grp in range(CH // 32):
            va = idx_s[pl.ds(grp * 16, 16)]
            vb = idx_s[pl.ds(CH // 2 + grp * 16, 16)]
            dstA = iota2 + (b * CH + grp * 32)
            plsc.store_scatter(idx_v, [dstA], va)
            plsc.store_scatter(idx_v, [dstA + 1], vb)
        for j in range(K):
            pltpu.async_copy(
                tlin_hbm.at[idx_v.at[pl.ds(b * CH + j * IPS, IPS)]],
                rows_v.at[pl.ds(b * CH + j * IPS, IPS)],
                gsem,
            )

    def wait_gathers(b):
        pltpu.make_async_copy(
            out_hbm.at[pl.ds(base, CH)], rows_v.at[pl.ds(b * CH, CH)], gsem
        ).wait()

    def store(g, b):
        off = base + g * CH
        pltpu.async_copy(
            rows_v.at[pl.ds(b * CH, CH)], out_hbm.at[pl.ds(off, CH)], osem
        )

    def wait_store():
        pltpu.make_async_copy(
            rows_v.at[pl.ds(0, CH)], out_hbm.at[pl.ds(base, CH)], osem
        ).wait()

    fire(0, 0)

    def body(g, carry):
        b = lax.rem(g, NBUF)
        pb = 1 - b

        @pl.when(g >= 2)
        def _():
            wait_store()

        fire(g, b)
        wait_gathers(pb)
        store(g - 1, pb)
        return carry

    lax.fori_loop(1, NCHUNK, body, 0)

    last = NCHUNK - 1
    wait_gathers(last % NBUF)
    store(last, last % NBUF)
    wait_store()
    wait_store()


_BB = 2048
_NBB = BATCH // _BB


def _fold_body(lin_ref, out_ref):
    a = lin_ref[...]
    c = jnp.concatenate([a[:, :D], a[:, D:]], axis=0)
    out_ref[...] = c.T[None, :, :]


_fold = pl.pallas_call(
    _fold_body,
    grid=(HIST, _NBB),
    in_specs=[pl.BlockSpec((_BB // 2, 2 * D), lambda h, b: (h * _NBB + b, 0))],
    out_specs=pl.BlockSpec((1, D, _BB), lambda h, b: (h, 0, b)),
    out_shape=jax.ShapeDtypeStruct((HIST, D, BATCH), jnp.float32),
)


def kernel(x, table):
    tpack = _relayout(jnp.swapaxes(table, 0, 1))
    tlin = tpack.reshape(2 * _PROWS, D)
    xT = jnp.swapaxes(x, 0, 1)
    xg = ((xT >> 12) << 12) + 2 * (xT & 2047) + ((xT >> 11) & 1)
    lin = _emb_gather(xg.reshape(B), tlin)
    out_phys = _fold(lin.reshape(B // 2, 2 * D))
    return jnp.transpose(out_phys, (2, 0, 1))


# speedup vs baseline: 2.3511x; 1.2311x over previous
import functools

import jax
import jax.numpy as jnp
from jax import lax
from jax.experimental import pallas as pl
from jax.experimental.pallas import tpu as pltpu
from jax.experimental.pallas import tpu_sc as plsc

VOCAB = 1_000_000
D = 64
BATCH = 16384
HIST = 50
B = BATCH * HIST

NC = 2
NS = 16
NW = NC * NS
BPW = B // NW

IPS = 128
K = 4
CH = K * IPS
NCHUNK = BPW // CH
NBUF = 2
FB = 4096


_W = 8192
_TGRID = -(-VOCAB // _W)
_PROWS = _TGRID * (_W // 2)


def _relayout_body(tin_ref, tout_ref):
    a = tin_ref[...]
    tout_ref[...] = jnp.concatenate(
        [a[:, : _W // 2].T, a[:, _W // 2 :].T], axis=1
    )


_relayout = pl.pallas_call(
    _relayout_body,
    grid=(_TGRID,),
    in_specs=[pl.BlockSpec((D, _W), lambda i: (0, i))],
    out_specs=pl.BlockSpec((_W // 2, 2 * D), lambda i: (i, 0)),
    out_shape=jax.ShapeDtypeStruct((_PROWS, 2 * D), jnp.float32),
)


_mesh = plsc.VectorSubcoreMesh(core_axis_name="c", subcore_axis_name="s")


@functools.partial(
    pl.kernel,
    mesh=_mesh,
    out_type=jax.ShapeDtypeStruct((B, D), jnp.float32),
    scratch_types=[
        pltpu.VMEM((NBUF * CH,), jnp.int32),
        pltpu.VMEM((CH,), jnp.int32),
        pltpu.VMEM((NBUF * CH, D), jnp.float32),
        pltpu.SemaphoreType.DMA,
        pltpu.SemaphoreType.DMA,
    ],
    compiler_params=pltpu.CompilerParams(
        use_tc_tiling_on_sc=False, needs_layout_passes=False
    ),
)
def _emb_gather(idx_hbm, tlin_hbm, out_hbm, idx_v, idx_s, rows_v, gsem, osem):
    wid = lax.axis_index("s") * NC + lax.axis_index("c")
    base = wid * BPW
    iota2 = lax.iota(jnp.int32, 16) * 2

    def fire(g, b):
        off = base + g * CH
        blk = off // FB
        pos = lax.rem(off, FB) // 2
        srcA = pl.multiple_of(blk * FB + pos, 256)
        pltpu.sync_copy(idx_hbm.at[pl.ds(srcA, CH // 2)], idx_s.at[pl.ds(0, CH // 2)])
        pltpu.sync_copy(
            idx_hbm.at[pl.ds(srcA + FB // 2, CH // 2)],
            idx_s.at[pl.ds(CH // 2, CH // 2)],
        )
        for grp in range(CH // 32):
            va = idx_s[pl.ds(grp * 16, 16)]
            vb = idx_s[pl.ds(CH // 2 + grp * 16, 16)]
            dstA = iota2 + (b * CH + grp * 32)
            plsc.store_scatter(idx_v, [dstA], va)
            plsc.store_scatter(idx_v, [dstA + 1], vb)
        for j in range(K):
            pltpu.async_copy(
                tlin_hbm.at[idx_v.at[pl.ds(b * CH + j * IPS, IPS)]],
                rows_v.at[pl.ds(b * CH + j * IPS, IPS)],
                gsem,
            )

    def wait_gathers(b):
        pltpu.make_async_copy(
            out_hbm.at[pl.ds(base, CH)], rows_v.at[pl.ds(b * CH, CH)], gsem
        ).wait()

    def store(g, b):
        off = base + g * CH
        pltpu.async_copy(
            rows_v.at[pl.ds(b * CH, CH)], out_hbm.at[pl.ds(off, CH)], osem
        )

    def wait_store():
        pltpu.make_async_copy(
            rows_v.at[pl.ds(0, CH)], out_hbm.at[pl.ds(base, CH)], osem
        ).wait()

    fire(0, 0)

    def body(g, carry):
        b = lax.rem(g, NBUF)
        pb = 1 - b

        @pl.when(g >= 2)
        def _():
            wait_store()

        fire(g, b)
        wait_gathers(pb)
        store(g - 1, pb)
        return carry

    lax.fori_loop(1, NCHUNK, body, 0)

    last = NCHUNK - 1
    wait_gathers(last % NBUF)
    store(last, last % NBUF)
    wait_store()
    wait_store()


_BB = 4096
_NBB = BATCH // _BB


def _fold_body(lin_ref, out_ref):
    a = lin_ref[...]
    c = jnp.concatenate([a[:, :D], a[:, D:]], axis=0)
    out_ref[...] = c.T[None, :, :]


_fold = pl.pallas_call(
    _fold_body,
    grid=(HIST, _NBB),
    in_specs=[pl.BlockSpec((_BB // 2, 2 * D), lambda h, b: (h * _NBB + b, 0))],
    out_specs=pl.BlockSpec((1, D, _BB), lambda h, b: (h, 0, b)),
    out_shape=jax.ShapeDtypeStruct((HIST, D, BATCH), jnp.float32),
)


def kernel(x, table):
    tpack = _relayout(jnp.swapaxes(table, 0, 1))
    tlin = tpack.reshape(2 * _PROWS, D)
    xT = jnp.swapaxes(x, 0, 1)
    xg = ((xT >> 13) << 13) + 2 * (xT & (_W // 2 - 1)) + ((xT >> 12) & 1)
    lin = _emb_gather(xg.reshape(B), tlin)
    out_phys = _fold(lin.reshape(B // 2, 2 * D))
    return jnp.transpose(out_phys, (2, 0, 1))


# speedup vs baseline: 2.6785x; 1.1393x over previous
import functools

import jax
import jax.numpy as jnp
from jax import lax
from jax.experimental import pallas as pl
from jax.experimental.pallas import tpu as pltpu
from jax.experimental.pallas import tpu_sc as plsc

VOCAB = 1_000_000
D = 64
BATCH = 16384
HIST = 50
B = BATCH * HIST

NC = 2
NS = 16
NW = NC * NS
BPW = B // NW

IPS = 128
K = 4
CH = K * IPS
NCHUNK = BPW // CH
NBUF = 2
FB = 8192


_W = 16384
_TGRID = -(-VOCAB // _W)
_PROWS = _TGRID * (_W // 2)


def _relayout_body(tin_ref, tout_ref):
    a = tin_ref[...]
    tout_ref[...] = jnp.concatenate(
        [a[:, : _W // 2].T, a[:, _W // 2 :].T], axis=1
    )


_relayout = pl.pallas_call(
    _relayout_body,
    grid=(_TGRID,),
    in_specs=[pl.BlockSpec((D, _W), lambda i: (0, i))],
    out_specs=pl.BlockSpec((_W // 2, 2 * D), lambda i: (i, 0)),
    out_shape=jax.ShapeDtypeStruct((_PROWS, 2 * D), jnp.float32),
)


_mesh = plsc.VectorSubcoreMesh(core_axis_name="c", subcore_axis_name="s")


@functools.partial(
    pl.kernel,
    mesh=_mesh,
    out_type=jax.ShapeDtypeStruct((B, D), jnp.float32),
    scratch_types=[
        pltpu.VMEM((NBUF * CH,), jnp.int32),
        pltpu.VMEM((CH,), jnp.int32),
        pltpu.VMEM((NBUF * CH, D), jnp.float32),
        pltpu.SemaphoreType.DMA,
        pltpu.SemaphoreType.DMA,
    ],
    compiler_params=pltpu.CompilerParams(
        use_tc_tiling_on_sc=False, needs_layout_passes=False
    ),
)
def _emb_gather(idx_hbm, tlin_hbm, out_hbm, idx_v, idx_s, rows_v, gsem, osem):
    wid = lax.axis_index("s") * NC + lax.axis_index("c")
    base = wid * BPW
    iota2 = lax.iota(jnp.int32, 16) * 2

    def fire(g, b):
        off = base + g * CH
        blk = off // FB
        pos = lax.rem(off, FB) // 2
        srcA = pl.multiple_of(blk * FB + pos, 256)
        pltpu.sync_copy(idx_hbm.at[pl.ds(srcA, CH // 2)], idx_s.at[pl.ds(0, CH // 2)])
        pltpu.sync_copy(
            idx_hbm.at[pl.ds(srcA + FB // 2, CH // 2)],
            idx_s.at[pl.ds(CH // 2, CH // 2)],
        )
        for grp in range(CH // 32):
            va = idx_s[pl.ds(grp * 16, 16)]
            vb = idx_s[pl.ds(CH // 2 + grp * 16, 16)]
            dstA = iota2 + (b * CH + grp * 32)
            plsc.store_scatter(idx_v, [dstA], va)
            plsc.store_scatter(idx_v, [dstA + 1], vb)
        for j in range(K):
            pltpu.async_copy(
                tlin_hbm.at[idx_v.at[pl.ds(b * CH + j * IPS, IPS)]],
                rows_v.at[pl.ds(b * CH + j * IPS, IPS)],
                gsem,
            )

    def wait_gathers(b):
        pltpu.make_async_copy(
            out_hbm.at[pl.ds(base, CH)], rows_v.at[pl.ds(b * CH, CH)], gsem
        ).wait()

    def store(g, b):
        off = base + g * CH
        pltpu.async_copy(
            rows_v.at[pl.ds(b * CH, CH)], out_hbm.at[pl.ds(off, CH)], osem
        )

    def wait_store():
        pltpu.make_async_copy(
            rows_v.at[pl.ds(0, CH)], out_hbm.at[pl.ds(base, CH)], osem
        ).wait()

    fire(0, 0)

    def body(g, carry):
        b = lax.rem(g, NBUF)
        pb = 1 - b

        @pl.when(g >= 2)
        def _():
            wait_store()

        fire(g, b)
        wait_gathers(pb)
        store(g - 1, pb)
        return carry

    lax.fori_loop(1, NCHUNK, body, 0)

    last = NCHUNK - 1
    wait_gathers(last % NBUF)
    store(last, last % NBUF)
    wait_store()
    wait_store()


_BB = 8192
_NBB = BATCH // _BB


def _fold_body(lin_ref, out_ref):
    a = lin_ref[...]
    c = jnp.concatenate([a[:, :D], a[:, D:]], axis=0)
    out_ref[...] = c.T[None, :, :]


_fold = pl.pallas_call(
    _fold_body,
    grid=(HIST, _NBB),
    in_specs=[pl.BlockSpec((_BB // 2, 2 * D), lambda h, b: (h * _NBB + b, 0))],
    out_specs=pl.BlockSpec((1, D, _BB), lambda h, b: (h, 0, b)),
    out_shape=jax.ShapeDtypeStruct((HIST, D, BATCH), jnp.float32),
)


def kernel(x, table):
    tpack = _relayout(jnp.swapaxes(table, 0, 1))
    tlin = tpack.reshape(2 * _PROWS, D)
    xT = jnp.swapaxes(x, 0, 1)
    xg = ((xT >> 14) << 14) + 2 * (xT & (_W // 2 - 1)) + ((xT >> 13) & 1)
    lin = _emb_gather(xg.reshape(B), tlin)
    out_phys = _fold(lin.reshape(B // 2, 2 * D))
    return jnp.transpose(out_phys, (2, 0, 1))


# speedup vs baseline: 2.8730x; 1.0726x over previous
import functools

import jax
import jax.numpy as jnp
from jax import lax
from jax.experimental import pallas as pl
from jax.experimental.pallas import tpu as pltpu
from jax.experimental.pallas import tpu_sc as plsc

VOCAB = 1_000_000
D = 64
BATCH = 16384
HIST = 50
B = BATCH * HIST

NC = 2
NS = 16
NW = NC * NS
BPW = B // NW

IPS = 128
K = 4
CH = K * IPS
NCHUNK = BPW // CH
NBUF = 2
FB = 16384


_W = 32768
_TGRID = -(-VOCAB // _W)
_PROWS = _TGRID * (_W // 2)


def _relayout_body(tin_ref, tout_ref):
    a = tin_ref[...]
    tout_ref[...] = jnp.concatenate(
        [a[:, : _W // 2].T, a[:, _W // 2 :].T], axis=1
    )


_relayout = pl.pallas_call(
    _relayout_body,
    grid=(_TGRID,),
    in_specs=[pl.BlockSpec((D, _W), lambda i: (0, i))],
    out_specs=pl.BlockSpec((_W // 2, 2 * D), lambda i: (i, 0)),
    out_shape=jax.ShapeDtypeStruct((_PROWS, 2 * D), jnp.float32),
)


_mesh = plsc.VectorSubcoreMesh(core_axis_name="c", subcore_axis_name="s")


@functools.partial(
    pl.kernel,
    mesh=_mesh,
    out_type=jax.ShapeDtypeStruct((B, D), jnp.float32),
    scratch_types=[
        pltpu.VMEM((NBUF * CH,), jnp.int32),
        pltpu.VMEM((CH,), jnp.int32),
        pltpu.VMEM((NBUF * CH, D), jnp.float32),
        pltpu.SemaphoreType.DMA,
        pltpu.SemaphoreType.DMA,
    ],
    compiler_params=pltpu.CompilerParams(
        use_tc_tiling_on_sc=False, needs_layout_passes=False
    ),
)
def _emb_gather(idx_hbm, tlin_hbm, out_hbm, idx_v, idx_s, rows_v, gsem, osem):
    wid = lax.axis_index("s") * NC + lax.axis_index("c")
    base = wid * BPW
    iota2 = lax.iota(jnp.int32, 16) * 2

    def fire(g, b):
        off = base + g * CH
        blk = off // FB
        pos = lax.rem(off, FB) // 2
        srcA = pl.multiple_of(blk * FB + pos, 256)
        pltpu.sync_copy(idx_hbm.at[pl.ds(srcA, CH // 2)], idx_s.at[pl.ds(0, CH // 2)])
        pltpu.sync_copy(
            idx_hbm.at[pl.ds(srcA + FB // 2, CH // 2)],
            idx_s.at[pl.ds(CH // 2, CH // 2)],
        )
        for grp in range(CH // 32):
            va = idx_s[pl.ds(grp * 16, 16)]
            vb = idx_s[pl.ds(CH // 2 + grp * 16, 16)]
            dstA = iota2 + (b * CH + grp * 32)
            plsc.store_scatter(idx_v, [dstA], va)
            plsc.store_scatter(idx_v, [dstA + 1], vb)
        for j in range(K):
            pltpu.async_copy(
                tlin_hbm.at[idx_v.at[pl.ds(b * CH + j * IPS, IPS)]],
                rows_v.at[pl.ds(b * CH + j * IPS, IPS)],
                gsem,
            )

    def wait_gathers(b):
        pltpu.make_async_copy(
            out_hbm.at[pl.ds(base, CH)], rows_v.at[pl.ds(b * CH, CH)], gsem
        ).wait()

    def store(g, b):
        off = base + g * CH
        pltpu.async_copy(
            rows_v.at[pl.ds(b * CH, CH)], out_hbm.at[pl.ds(off, CH)], osem
        )

    def wait_store():
        pltpu.make_async_copy(
            rows_v.at[pl.ds(0, CH)], out_hbm.at[pl.ds(base, CH)], osem
        ).wait()

    fire(0, 0)

    def body(g, carry):
        b = lax.rem(g, NBUF)
        pb = 1 - b

        @pl.when(g >= 2)
        def _():
            wait_store()

        fire(g, b)
        wait_gathers(pb)
        store(g - 1, pb)
        return carry

    lax.fori_loop(1, NCHUNK, body, 0)

    last = NCHUNK - 1
    wait_gathers(last % NBUF)
    store(last, last % NBUF)
    wait_store()
    wait_store()


_BB = 16384
_NBB = BATCH // _BB


def _fold_body(lin_ref, out_ref):
    a = lin_ref[...]
    c = jnp.concatenate([a[:, :D], a[:, D:]], axis=0)
    out_ref[...] = c.T[None, :, :]


_fold = pl.pallas_call(
    _fold_body,
    grid=(HIST, _NBB),
    in_specs=[pl.BlockSpec((_BB // 2, 2 * D), lambda h, b: (h * _NBB + b, 0))],
    out_specs=pl.BlockSpec((1, D, _BB), lambda h, b: (h, 0, b)),
    out_shape=jax.ShapeDtypeStruct((HIST, D, BATCH), jnp.float32),
)


def kernel(x, table):
    tpack = _relayout(jnp.swapaxes(table, 0, 1))
    tlin = tpack.reshape(2 * _PROWS, D)
    xT = jnp.swapaxes(x, 0, 1)
    xg = ((xT >> 15) << 15) + 2 * (xT & (_W // 2 - 1)) + ((xT >> 14) & 1)
    lin = _emb_gather(xg.reshape(B), tlin)
    out_phys = _fold(lin.reshape(B // 2, 2 * D))
    return jnp.transpose(out_phys, (2, 0, 1))
